# baseline (speedup 1.0000x reference)
import functools

import jax
import jax.numpy as jnp
from jax import lax
from jax.experimental import pallas as pl
from jax.experimental.pallas import tpu as pltpu
from jax.experimental.pallas import tpu_sc as plsc

V = 1000
D = 1000
B, T = 1024, 200
NC, NS = 2, 16
NW = NC * NS
CG = D // 8
GPW = 4
NBV = B // 128


def _sc_gather_t(idx_t, table_tp):
    mesh = plsc.VectorSubcoreMesh(core_axis_name="c", subcore_axis_name="s")

    @functools.partial(
        pl.kernel,
        mesh=mesh,
        out_type=jax.ShapeDtypeStruct((T, CG, 8, 8, 128), jnp.float32),
        scratch_types=[
            pltpu.VMEM((GPW * 8, V), jnp.float32),
            pltpu.VMEM((B,), jnp.int32),
            pltpu.VMEM((8, 8, 128), jnp.float32),
            pltpu.VMEM((8, 8, 128), jnp.float32),
            pltpu.SemaphoreType.DMA,
            pltpu.SemaphoreType.DMA,
        ],
        compiler_params=pltpu.CompilerParams(
            use_tc_tiling_on_sc=False, needs_layout_passes=False
        ),
    )
    def k(idx_hbm, table_hbm, out_hbm, tabv, idxv, slab0, slab1, semw0, semw1):
        wid = lax.axis_index("s") * NC + lax.axis_index("c")
        c0 = wid * (GPW * 8)
        pltpu.sync_copy(table_hbm.at[pl.ds(c0, GPW * 8)], tabv)

        slabs = (slab0, slab1)
        sems = (semw0, semw1)

        def build(g, slab):
            for bg in range(NBV):
                idx16 = [
                    idxv[pl.ds((bg * 8 + kk) * 16, 16)] for kk in range(8)
                ]
                for c8 in range(8):
                    row = jnp.full((16,), g * 8 + c8, dtype=jnp.int32)
                    for kk in range(8):
                        vals = plsc.load_gather(tabv, [row, idx16[kk]])
                        slab[bg, c8, pl.ds(kk * 16, 16)] = vals

        def tbody(t, carry):
            pltpu.sync_copy(idx_hbm.at[t], idxv)
            for g in range(GPW):
                p = g % 2
                g_abs = wid * GPW + g

                @pl.when(g_abs < CG)
                def _():
                    if g < 2:
                        @pl.when(t > 0)
                        def _():
                            pltpu.make_async_copy(
                                slabs[p], out_hbm.at[0, 0], sems[p]
                            ).wait()
                    else:
                        pltpu.make_async_copy(
                            slabs[p], out_hbm.at[0, 0], sems[p]
                        ).wait()
                    build(g, slabs[p])
                    pltpu.async_copy(
                        slabs[p], out_hbm.at[t, g_abs], sems[p]
                    )
            return carry

        lax.fori_loop(0, T, tbody, 0)
        pltpu.make_async_copy(slab0, out_hbm.at[0, 0], semw0).wait()

        @pl.when(wid * GPW + 1 < CG)
        def _():
            pltpu.make_async_copy(slab1, out_hbm.at[0, 0], semw1).wait()

    return k(idx_t, table_tp)


def kernel(idx, token_embedding_table):
    idx_t = idx.astype(jnp.int32).T
    table_tp = jnp.pad(token_embedding_table.T, ((0, NW * GPW * 8 - D), (0, 0)))
    out5 = _sc_gather_t(idx_t, table_tp)
    return out5.transpose(2, 4, 0, 1, 3).reshape(B, T, D)


# speedup vs baseline: 1.5437x; 1.5437x over previous
import functools

import jax
import jax.numpy as jnp
from jax import lax
from jax.experimental import pallas as pl
from jax.experimental.pallas import tpu as pltpu
from jax.experimental.pallas import tpu_sc as plsc

V = 1000
D = 1000
B, T = 1024, 200
NC, NS = 2, 16
NW = NC * NS
CG = D // 8
GPW = 4
NBV = B // 128


def _sc_gather_t(idx_t, table_tp):
    mesh = plsc.VectorSubcoreMesh(core_axis_name="c", subcore_axis_name="s")

    @functools.partial(
        pl.kernel,
        mesh=mesh,
        out_type=jax.ShapeDtypeStruct((T, CG, 8, 8, 128), jnp.float32),
        scratch_types=[
            pltpu.VMEM((GPW * 8, V), jnp.float32),
            pltpu.VMEM((B,), jnp.int32),
            pltpu.VMEM((8, 8, 128), jnp.float32),
            pltpu.VMEM((8, 8, 128), jnp.float32),
            pltpu.SemaphoreType.DMA,
            pltpu.SemaphoreType.DMA,
        ],
        compiler_params=pltpu.CompilerParams(
            use_tc_tiling_on_sc=False, needs_layout_passes=False
        ),
    )
    def k(idx_hbm, table_hbm, out_hbm, tabv, idxv, slab0, slab1, semw0, semw1):
        wid = lax.axis_index("s") * NC + lax.axis_index("c")
        c0 = wid * (GPW * 8)
        pltpu.sync_copy(table_hbm.at[pl.ds(c0, GPW * 8)], tabv)

        slabs = (slab0, slab1)
        sems = (semw0, semw1)

        def build(g, slab):
            def bgbody(bg, carry):
                idx16 = [
                    idxv[pl.ds(bg * 128 + kk * 16, 16)] for kk in range(8)
                ]
                for c8 in range(8):
                    row = jnp.full((16,), g * 8 + c8, dtype=jnp.int32)
                    for kk in range(8):
                        vals = plsc.load_gather(tabv, [row, idx16[kk]])
                        slab[bg, c8, pl.ds(kk * 16, 16)] = vals
                return carry

            lax.fori_loop(0, NBV, bgbody, 0)

        def tbody(t, carry):
            pltpu.sync_copy(idx_hbm.at[t], idxv)
            for g in range(GPW):
                p = g % 2
                g_abs = wid * GPW + g

                @pl.when(g_abs < CG)
                def _():
                    if g < 2:
                        @pl.when(t > 0)
                        def _():
                            pltpu.make_async_copy(
                                slabs[p], out_hbm.at[0, 0], sems[p]
                            ).wait()
                    else:
                        pltpu.make_async_copy(
                            slabs[p], out_hbm.at[0, 0], sems[p]
                        ).wait()
                    build(g, slabs[p])
                    pltpu.async_copy(
                        slabs[p], out_hbm.at[t, g_abs], sems[p]
                    )
            return carry

        lax.fori_loop(0, T, tbody, 0)
        pltpu.make_async_copy(slab0, out_hbm.at[0, 0], semw0).wait()

        @pl.when(wid * GPW + 1 < CG)
        def _():
            pltpu.make_async_copy(slab1, out_hbm.at[0, 0], semw1).wait()

    return k(idx_t, table_tp)


def kernel(idx, token_embedding_table):
    idx_t = idx.astype(jnp.int32).T
    table_tp = jnp.pad(token_embedding_table.T, ((0, NW * GPW * 8 - D), (0, 0)))
    out5 = _sc_gather_t(idx_t, table_tp)
    return out5.transpose(2, 4, 0, 1, 3).reshape(B, T, D)


# speedup vs baseline: 2.4664x; 1.5977x over previous
import functools

import jax
import jax.numpy as jnp
from jax import lax
from jax.experimental import pallas as pl
from jax.experimental.pallas import tpu as pltpu
from jax.experimental.pallas import tpu_sc as plsc

V = 1000
D = 1000
NT = 7
TAIL = D - NT * 128
BT = 1024 * 200
NC, NS = 2, 16
NW = NC * NS
B_PER_W = BT // NW
CHUNK = 40
N_CHUNKS = B_PER_W // CHUNK
TAIL_COPIES = tuple(
    (128 - TAIL + k * 16, NT * 128 + k * 16) for k in range(TAIL // 16)
)


def _sc_gather(idx_flat, table, table_tail):
    mesh = plsc.VectorSubcoreMesh(core_axis_name="c", subcore_axis_name="s")

    @functools.partial(
        pl.kernel,
        mesh=mesh,
        out_type=jax.ShapeDtypeStruct((BT, D), jnp.float32),
        scratch_types=[
            pltpu.VMEM((B_PER_W,), jnp.int32),
            pltpu.VMEM((CHUNK, D), jnp.float32),
            pltpu.VMEM((CHUNK, D), jnp.float32),
            pltpu.VMEM((CHUNK, 128), jnp.float32),
            pltpu.VMEM((CHUNK, 128), jnp.float32),
            pltpu.SemaphoreType.DMA,
            pltpu.SemaphoreType.DMA,
        ],
        compiler_params=pltpu.CompilerParams(needs_layout_passes=False),
    )
    def k(idx_hbm, table_hbm, tail_hbm, out_hbm,
          idx_v, rows_a, rows_b, last_a, last_b, sem_g, sem_w):
        wid = lax.axis_index("s") * NC + lax.axis_index("c")
        base = wid * B_PER_W
        pltpu.sync_copy(idx_hbm.at[pl.ds(base, B_PER_W)], idx_v)

        def gathers(g, rows, last):
            s = idx_v.at[pl.ds(g * CHUNK, CHUNK)]
            for j in range(NT):
                pltpu.async_copy(
                    table_hbm.at[s, pl.ds(j * 128, 128)],
                    rows.at[:, pl.ds(j * 128, 128)],
                    sem_g,
                )
            pltpu.async_copy(tail_hbm.at[s], last, sem_g)

        def wait_gathers():
            for _ in range(NT + 1):
                pltpu.make_async_copy(
                    tail_hbm.at[idx_v.at[pl.ds(0, CHUNK)]], last_a, sem_g
                ).wait()

        def tailpass(rows, last):
            col_ids = lax.iota(jnp.int32, 16) + (D - 16)

            def rbody(r, carry):
                for src, dst in TAIL_COPIES:
                    rows[r, pl.ds(dst, 16)] = last[r, pl.ds(src, 16)]
                x = last[r, pl.ds(112, 16)]
                row_ids = jnp.full((16,), r, dtype=jnp.int32)
                plsc.store_scatter(rows, [row_ids, col_ids], x)
                return carry

            lax.fori_loop(0, CHUNK, rbody, 0)

        def wr(g, rows):
            pltpu.async_copy(
                rows, out_hbm.at[pl.ds(base + g * CHUNK, CHUNK)], sem_w
            )

        def wait_write():
            pltpu.make_async_copy(
                rows_a, out_hbm.at[pl.ds(base, CHUNK)], sem_w
            ).wait()

        gathers(0, rows_a, last_a)
        wait_gathers()
        tailpass(rows_a, last_a)
        wr(0, rows_a)
        gathers(1, rows_b, last_b)

        def body(h, carry):
            g = 2 * h + 1
            wait_gathers()
            wait_write()
            gathers(g + 1, rows_a, last_a)
            tailpass(rows_b, last_b)
            wr(g, rows_b)
            wait_gathers()
            wait_write()
            gathers(g + 2, rows_b, last_b)
            tailpass(rows_a, last_a)
            wr(g + 1, rows_a)
            return carry

        lax.fori_loop(0, (N_CHUNKS - 2) // 2, body, 0)
        wait_gathers()
        wait_write()
        tailpass(rows_b, last_b)
        wr(N_CHUNKS - 1, rows_b)
        wait_write()

    return k(idx_flat, table, table_tail)


def kernel(idx, token_embedding_table):
    idx_flat = idx.reshape(-1).astype(jnp.int32)
    table_tail = token_embedding_table[:, D - 128:]
    out = _sc_gather(idx_flat, token_embedding_table, table_tail)
    return out.reshape(idx.shape[0], idx.shape[1], D)
